# baseline (speedup 1.0000x reference)
import functools

import jax
import jax.numpy as jnp
from jax.experimental import pallas as pl
from jax.experimental.pallas import tpu as pltpu

K = 32


def _encode_body(x_ref, w_ref, benc_ref, bdec_ref, z_ref):
    xc = x_ref[...] - bdec_ref[...]
    acc = jax.lax.dot_general(
        xc, w_ref[...],
        dimension_numbers=(((1,), (1,)), ((), ())),
        preferred_element_type=jnp.float32,
    )
    z_ref[...] = jnp.maximum(acc + benc_ref[...], 0.0)


def _encode(x, W_enc, b_enc, b_dec, bn):
    B, D = x.shape
    N = W_enc.shape[0]
    grid = (N // bn,)
    return pl.pallas_call(
        _encode_body,
        grid=grid,
        in_specs=[
            pl.BlockSpec((B, D), lambda n: (0, 0)),
            pl.BlockSpec((bn, D), lambda n: (n, 0)),
            pl.BlockSpec((1, bn), lambda n: (0, n)),
            pl.BlockSpec((1, D), lambda n: (0, 0)),
        ],
        out_specs=pl.BlockSpec((B, bn), lambda n: (0, n)),
        out_shape=jax.ShapeDtypeStruct((B, N), jnp.float32),
    )(x, W_enc, b_enc.reshape(1, N), b_dec.reshape(1, D))


def _topk_body(z_ref, idx_ref, thr_ref, scratch_ref, k):
    bm, n = z_ref.shape
    scratch_ref[...] = z_ref[...]
    col = jax.lax.broadcasted_iota(jnp.int32, (bm, n), 1)
    colk = jax.lax.broadcasted_iota(jnp.int32, (bm, k), 1)
    idxs = jnp.zeros((bm, k), jnp.int32)
    m = None
    for it in range(k):
        zc = scratch_ref[...]
        m = jnp.max(zc, axis=1, keepdims=True)
        cand = jnp.where(zc == m, col, n)
        j = jnp.min(cand, axis=1, keepdims=True)
        idxs = jnp.where(colk == it, j, idxs)
        scratch_ref[...] = jnp.where(col == j, -zc - 1.0, zc)
    idx_ref[...] = idxs
    thr_ref[...] = m


def _topk(z_pre, bm, k):
    B, N = z_pre.shape
    grid = (B // bm,)
    return pl.pallas_call(
        functools.partial(_topk_body, k=k),
        grid=grid,
        in_specs=[pl.BlockSpec((bm, N), lambda r: (r, 0))],
        out_specs=[
            pl.BlockSpec((bm, k), lambda r: (r, 0)),
            pl.BlockSpec((bm, 1), lambda r: (r, 0)),
        ],
        out_shape=[
            jax.ShapeDtypeStruct((B, k), jnp.int32),
            jax.ShapeDtypeStruct((B, 1), jnp.float32),
        ],
        scratch_shapes=[pltpu.VMEM((bm, N), jnp.float32)],
    )(z_pre)


def _decode_body(z_ref, w_ref, thr_ref, x_ref, bdec_ref,
                 ztopk_ref, xhat_ref, loss_ref, *, nsteps, scale):
    r = pl.program_id(0)
    n = pl.program_id(1)
    nprog = pl.num_programs(0)
    zt = z_ref[...]
    zt = jnp.where(zt >= thr_ref[...], zt, 0.0)
    ztopk_ref[...] = zt
    acc = jax.lax.dot_general(
        zt, w_ref[...],
        dimension_numbers=(((1,), (1,)), ((), ())),
        preferred_element_type=jnp.float32,
    )

    @pl.when(n == 0)
    def _():
        xhat_ref[...] = acc

    @pl.when(n != 0)
    def _():
        xhat_ref[...] += acc

    @pl.when(n == nsteps - 1)
    def _():
        xh = xhat_ref[...] + bdec_ref[...]
        xhat_ref[...] = xh
        diff = xh - x_ref[...]
        part = (jnp.sum(diff * diff) * scale).reshape(1, 1)

        @pl.when(r == 0)
        def _():
            loss_ref[...] = part

        @pl.when(r != 0)
        def _():
            loss_ref[...] += part


def _decode(z_pre, W_dec, thr, x, b_dec, bm, bn):
    B, D = x.shape
    N = z_pre.shape[1]
    nsteps = N // bn
    grid = (B // bm, nsteps)
    return pl.pallas_call(
        functools.partial(_decode_body, nsteps=nsteps, scale=1.0 / (B * D)),
        grid=grid,
        in_specs=[
            pl.BlockSpec((bm, bn), lambda r, n: (r, n)),
            pl.BlockSpec((D, bn), lambda r, n: (0, n)),
            pl.BlockSpec((bm, 1), lambda r, n: (r, 0)),
            pl.BlockSpec((bm, D), lambda r, n: (r, 0)),
            pl.BlockSpec((1, D), lambda r, n: (0, 0)),
        ],
        out_specs=[
            pl.BlockSpec((bm, bn), lambda r, n: (r, n)),
            pl.BlockSpec((bm, D), lambda r, n: (r, 0)),
            pl.BlockSpec((1, 1), lambda r, n: (0, 0)),
        ],
        out_shape=[
            jax.ShapeDtypeStruct((B, N), jnp.float32),
            jax.ShapeDtypeStruct((B, D), jnp.float32),
            jax.ShapeDtypeStruct((1, 1), jnp.float32),
        ],
    )(z_pre, W_dec, thr, x, b_dec.reshape(1, D))


def kernel(x, W_enc, b_enc, W_dec, b_dec):
    B, D = x.shape
    N = W_enc.shape[0]
    bn_enc = min(512, N)
    bm_topk = min(128, B)
    bm_dec = min(1024, B)
    bn_dec = min(512, N)

    z_pre = _encode(x, W_enc, b_enc, b_dec, bn_enc)
    topk_idx, thr = _topk(z_pre, bm_topk, K)
    z_topk, x_hat, loss = _decode(z_pre, W_dec, thr, x, b_dec, bm_dec, bn_dec)
    return (topk_idx, thr)


# speedup vs baseline: 1.8817x; 1.8817x over previous
import functools

import jax
import jax.numpy as jnp
from jax import lax
from jax.experimental import pallas as pl
from jax.experimental.pallas import tpu as pltpu
from jax.experimental.pallas import tpu_sc as plsc

K = 32
CHUNK_W = 32
NUM_SC_WORKERS = 32


def _encode_body(x_ref, w_ref, benc_ref, bdec_ref, z_ref, cm_ref):
    xc = x_ref[...] - bdec_ref[...]
    acc = jax.lax.dot_general(
        xc, w_ref[...],
        dimension_numbers=(((1,), (1,)), ((), ())),
        preferred_element_type=jnp.float32,
    )
    z = jnp.maximum(acc + benc_ref[...], 0.0)
    z_ref[...] = z
    bn = z.shape[1]
    cpb = bn // CHUNK_W
    m = z
    s = CHUNK_W // 2
    while s >= 1:
        m = jnp.maximum(m, jnp.concatenate([m[:, s:], m[:, :s]], axis=1))
        s //= 2
    cm_tile = jnp.concatenate(
        [m[:, c * CHUNK_W:c * CHUNK_W + 1] for c in range(cpb)], axis=1)
    cm_ref[0, :, :] = cm_tile


def _encode(x, W_enc, b_enc, b_dec, bn):
    B, D = x.shape
    N = W_enc.shape[0]
    C = N // CHUNK_W
    cpb = bn // CHUNK_W
    grid = (N // bn,)
    return pl.pallas_call(
        _encode_body,
        grid=grid,
        in_specs=[
            pl.BlockSpec((B, D), lambda n: (0, 0)),
            pl.BlockSpec((bn, D), lambda n: (n, 0)),
            pl.BlockSpec((1, bn), lambda n: (0, n)),
            pl.BlockSpec((1, D), lambda n: (0, 0)),
        ],
        out_specs=[
            pl.BlockSpec((B, bn), lambda n: (0, n)),
            pl.BlockSpec((1, B, cpb), lambda n: (n, 0, 0)),
        ],
        out_shape=[
            jax.ShapeDtypeStruct((B, N), jnp.float32),
            jax.ShapeDtypeStruct((N // bn, B, cpb), jnp.float32),
        ],
    )(x, W_enc, b_enc.reshape(1, N), b_dec.reshape(1, D))


def _select_body(cm_ref, cid_ref, gat_ref, scratch_ref, *, C, bm):
    r = pl.program_id(0)
    scratch_ref[...] = cm_ref[...]
    colC = lax.broadcasted_iota(jnp.int32, (bm, C), 1)
    colk = lax.broadcasted_iota(jnp.int32, (bm, K), 1)
    rowk = lax.broadcasted_iota(jnp.int32, (bm, K), 0) + r * bm
    cids = jnp.zeros((bm, K), jnp.int32)
    for it in range(K):
        cmw = scratch_ref[...]
        m = jnp.max(cmw, axis=1, keepdims=True)
        cand = jnp.where(cmw == m, colC, C)
        j = jnp.min(cand, axis=1, keepdims=True)
        cids = jnp.where(colk == it, j, cids)
        scratch_ref[...] = jnp.where(colC == j, -1.0, cmw)
    cid_ref[...] = cids
    gat_ref[...] = rowk * C + cids


def _select(cm, bm):
    B, C = cm.shape
    grid = (B // bm,)
    return pl.pallas_call(
        functools.partial(_select_body, C=C, bm=bm),
        grid=grid,
        in_specs=[pl.BlockSpec((bm, C), lambda r: (r, 0))],
        out_specs=[
            pl.BlockSpec((bm, K), lambda r: (r, 0)),
            pl.BlockSpec((bm, K), lambda r: (r, 0)),
        ],
        out_shape=[
            jax.ShapeDtypeStruct((B, K), jnp.int32),
            jax.ShapeDtypeStruct((B, K), jnp.int32),
        ],
        scratch_shapes=[pltpu.VMEM((bm, C), jnp.float32)],
    )(cm)


def _sc_gather(ztab, gat):
    nidx = gat.shape[0]
    per_w = nidx // NUM_SC_WORKERS
    nch = per_w // 128
    idx3 = gat.reshape(NUM_SC_WORKERS, nch, 128)

    @functools.partial(
        pl.kernel,
        out_type=jax.ShapeDtypeStruct((NUM_SC_WORKERS, nch, 128, CHUNK_W),
                                      jnp.float32),
        mesh=plsc.VectorSubcoreMesh(core_axis_name="c", subcore_axis_name="s"),
        compiler_params=pltpu.CompilerParams(use_tc_tiling_on_sc=False),
        scratch_types=[
            pltpu.VMEM((nch, 128), jnp.int32),
            pltpu.VMEM((nch, 128, CHUNK_W), jnp.float32),
            pltpu.SemaphoreType.DMA,
        ],
    )
    def gk(ztab_hbm, idx_hbm, out_hbm, idx_v, rows_v, sem):
        wid = lax.axis_index("s") * 2 + lax.axis_index("c")
        pltpu.sync_copy(idx_hbm.at[wid], idx_v)
        copies = [pltpu.async_copy(ztab_hbm.at[idx_v.at[j]], rows_v.at[j], sem)
                  for j in range(nch)]
        for cp in copies:
            cp.wait()
        pltpu.sync_copy(rows_v, out_hbm.at[wid])

    out = gk(ztab, idx3)
    return out.reshape(nidx, CHUNK_W)


def _final_body(cand_ref, cid_ref, idx_ref, thr_ref, scratch_ref, *, bm):
    ncand = K * CHUNK_W
    scratch_ref[...] = cand_ref[...]
    col = lax.broadcasted_iota(jnp.int32, (bm, ncand), 1)
    colk = lax.broadcasted_iota(jnp.int32, (bm, K), 1)
    ek = lax.broadcasted_iota(jnp.int32, (K, ncand), 0)
    ej = lax.broadcasted_iota(jnp.int32, (K, ncand), 1) // CHUNK_W
    onehot = jnp.where(ek == ej, 1.0, 0.0).astype(jnp.float32)
    cid_exp = jax.lax.dot_general(
        cid_ref[...].astype(jnp.float32), onehot,
        dimension_numbers=(((1,), (0,)), ((), ())),
        preferred_element_type=jnp.float32,
        precision=jax.lax.Precision.HIGHEST,
    )
    gidx = cid_exp.astype(jnp.int32) * CHUNK_W + (col % CHUNK_W)
    idxs = jnp.zeros((bm, K), jnp.int32)
    m = None
    for it in range(K):
        v = scratch_ref[...]
        m = jnp.max(v, axis=1, keepdims=True)
        eq = v == m
        sel = jnp.where(eq, gidx, 1 << 30)
        g = jnp.min(sel, axis=1, keepdims=True)
        idxs = jnp.where(colk == it, g, idxs)
        scratch_ref[...] = jnp.where(eq & (gidx == g), -1.0, v)
    idx_ref[...] = idxs
    thr_ref[...] = m


def _final_topk(cand, cid, bm):
    B = cand.shape[0]
    ncand = K * CHUNK_W
    grid = (B // bm,)
    return pl.pallas_call(
        functools.partial(_final_body, bm=bm),
        grid=grid,
        in_specs=[
            pl.BlockSpec((bm, ncand), lambda r: (r, 0)),
            pl.BlockSpec((bm, K), lambda r: (r, 0)),
        ],
        out_specs=[
            pl.BlockSpec((bm, K), lambda r: (r, 0)),
            pl.BlockSpec((bm, 1), lambda r: (r, 0)),
        ],
        out_shape=[
            jax.ShapeDtypeStruct((B, K), jnp.int32),
            jax.ShapeDtypeStruct((B, 1), jnp.float32),
        ],
        scratch_shapes=[pltpu.VMEM((bm, ncand), jnp.float32)],
    )(cand, cid)


def _decode_body(z_ref, w_ref, thr_ref, x_ref, bdec_ref,
                 ztopk_ref, xhat_ref, loss_ref, *, nsteps, scale):
    r = pl.program_id(0)
    n = pl.program_id(1)
    zt = z_ref[...]
    zt = jnp.where(zt >= thr_ref[...], zt, 0.0)
    ztopk_ref[...] = zt
    acc = jax.lax.dot_general(
        zt, w_ref[...],
        dimension_numbers=(((1,), (1,)), ((), ())),
        preferred_element_type=jnp.float32,
    )

    @pl.when(n == 0)
    def _():
        xhat_ref[...] = acc

    @pl.when(n != 0)
    def _():
        xhat_ref[...] += acc

    @pl.when(n == nsteps - 1)
    def _():
        xh = xhat_ref[...] + bdec_ref[...]
        xhat_ref[...] = xh
        diff = xh - x_ref[...]
        part = (jnp.sum(diff * diff) * scale).reshape(1, 1)

        @pl.when(r == 0)
        def _():
            loss_ref[...] = part

        @pl.when(r != 0)
        def _():
            loss_ref[...] += part


def _decode(z_pre, W_dec, thr, x, b_dec, bm, bn):
    B, D = x.shape
    N = z_pre.shape[1]
    nsteps = N // bn
    grid = (B // bm, nsteps)
    return pl.pallas_call(
        functools.partial(_decode_body, nsteps=nsteps, scale=1.0 / (B * D)),
        grid=grid,
        in_specs=[
            pl.BlockSpec((bm, bn), lambda r, n: (r, n)),
            pl.BlockSpec((D, bn), lambda r, n: (0, n)),
            pl.BlockSpec((bm, 1), lambda r, n: (r, 0)),
            pl.BlockSpec((bm, D), lambda r, n: (r, 0)),
            pl.BlockSpec((1, D), lambda r, n: (0, 0)),
        ],
        out_specs=[
            pl.BlockSpec((bm, bn), lambda r, n: (r, n)),
            pl.BlockSpec((bm, D), lambda r, n: (r, 0)),
            pl.BlockSpec((1, 1), lambda r, n: (0, 0)),
        ],
        out_shape=[
            jax.ShapeDtypeStruct((B, N), jnp.float32),
            jax.ShapeDtypeStruct((B, D), jnp.float32),
            jax.ShapeDtypeStruct((1, 1), jnp.float32),
        ],
    )(z_pre, W_dec, thr, x, b_dec.reshape(1, D))


def kernel(x, W_enc, b_enc, W_dec, b_dec):
    B, D = x.shape
    N = W_enc.shape[0]
    C = N // CHUNK_W

    z_pre, cm3 = _encode(x, W_enc, b_enc, b_dec, min(512, N))
    cm = jnp.transpose(cm3, (1, 0, 2)).reshape(B, C)
    cid, gat = _select(cm, min(1024, B))
    cand = _sc_gather(z_pre.reshape(B * C, CHUNK_W), gat.reshape(B * K))
    topk_idx, thr = _final_topk(cand.reshape(B, K * CHUNK_W), cid, min(512, B))
    z_topk, x_hat, loss = _decode(z_pre, W_dec, thr, x, b_dec,
                                  min(1024, B), min(512, N))
    return (x_hat, z_topk, z_pre, topk_idx, loss.reshape(()))


# speedup vs baseline: 2.3205x; 1.2332x over previous
import functools

import jax
import jax.numpy as jnp
from jax import lax
from jax.experimental import pallas as pl
from jax.experimental.pallas import tpu as pltpu
from jax.experimental.pallas import tpu_sc as plsc

K = 32
CHUNK_W = 32
NUM_SC_WORKERS = 32


def _encode_body(x_ref, w_ref, benc_ref, bdec_ref, z_ref, cm_ref):
    xc = x_ref[...] - bdec_ref[...]
    acc = jax.lax.dot_general(
        xc, w_ref[...],
        dimension_numbers=(((1,), (1,)), ((), ())),
        preferred_element_type=jnp.float32,
    )
    z = jnp.maximum(acc + benc_ref[...], 0.0)
    z_ref[...] = z
    bn = z.shape[1]
    cpb = bn // CHUNK_W
    cm_tile = jnp.concatenate(
        [jnp.max(z[:, c * CHUNK_W:(c + 1) * CHUNK_W], axis=1, keepdims=True)
         for c in range(cpb)], axis=1)
    cm_ref[0, :, :] = cm_tile


def _encode(x, W_enc, b_enc, b_dec, bn):
    B, D = x.shape
    N = W_enc.shape[0]
    C = N // CHUNK_W
    cpb = bn // CHUNK_W
    grid = (N // bn,)
    return pl.pallas_call(
        _encode_body,
        grid=grid,
        in_specs=[
            pl.BlockSpec((B, D), lambda n: (0, 0)),
            pl.BlockSpec((bn, D), lambda n: (n, 0)),
            pl.BlockSpec((1, bn), lambda n: (0, n)),
            pl.BlockSpec((1, D), lambda n: (0, 0)),
        ],
        out_specs=[
            pl.BlockSpec((B, bn), lambda n: (0, n)),
            pl.BlockSpec((1, B, cpb), lambda n: (n, 0, 0)),
        ],
        out_shape=[
            jax.ShapeDtypeStruct((B, N), jnp.float32),
            jax.ShapeDtypeStruct((N // bn, B, cpb), jnp.float32),
        ],
    )(x, W_enc, b_enc.reshape(1, N), b_dec.reshape(1, D))


def _select_body(cm_ref, cid_ref, gat_ref, scratch_ref, *, C, bm):
    r = pl.program_id(0)
    scratch_ref[...] = cm_ref[...]
    colC = lax.broadcasted_iota(jnp.int32, (bm, C), 1)
    colk = lax.broadcasted_iota(jnp.int32, (bm, K), 1)
    rowk = lax.broadcasted_iota(jnp.int32, (bm, K), 0) + r * bm
    cids = jnp.zeros((bm, K), jnp.int32)
    for it in range(K):
        cmw = scratch_ref[...]
        m = jnp.max(cmw, axis=1, keepdims=True)
        cand = jnp.where(cmw == m, colC, C)
        j = jnp.min(cand, axis=1, keepdims=True)
        cids = jnp.where(colk == it, j, cids)
        scratch_ref[...] = jnp.where(colC == j, -1.0, cmw)
    cid_ref[...] = cids
    gat_ref[...] = rowk * C + cids


def _select(cm, bm):
    B, C = cm.shape
    grid = (B // bm,)
    return pl.pallas_call(
        functools.partial(_select_body, C=C, bm=bm),
        grid=grid,
        in_specs=[pl.BlockSpec((bm, C), lambda r: (r, 0))],
        out_specs=[
            pl.BlockSpec((bm, K), lambda r: (r, 0)),
            pl.BlockSpec((bm, K), lambda r: (r, 0)),
        ],
        out_shape=[
            jax.ShapeDtypeStruct((B, K), jnp.int32),
            jax.ShapeDtypeStruct((B, K), jnp.int32),
        ],
        scratch_shapes=[pltpu.VMEM((bm, C), jnp.float32)],
    )(cm)


def _sc_gather(ztab, gat):
    nidx = gat.shape[0]
    per_w = nidx // NUM_SC_WORKERS
    nch = per_w // 128
    idx3 = gat.reshape(NUM_SC_WORKERS, nch, 128)

    @functools.partial(
        pl.kernel,
        out_type=jax.ShapeDtypeStruct((NUM_SC_WORKERS, nch, 128, CHUNK_W),
                                      jnp.float32),
        mesh=plsc.VectorSubcoreMesh(core_axis_name="c", subcore_axis_name="s"),
        compiler_params=pltpu.CompilerParams(use_tc_tiling_on_sc=False),
        scratch_types=[
            pltpu.VMEM((nch, 128), jnp.int32),
            pltpu.VMEM((nch, 128, CHUNK_W), jnp.float32),
            pltpu.SemaphoreType.DMA,
        ],
    )
    def gk(ztab_hbm, idx_hbm, out_hbm, idx_v, rows_v, sem):
        wid = lax.axis_index("s") * 2 + lax.axis_index("c")
        pltpu.sync_copy(idx_hbm.at[wid], idx_v)
        copies = [pltpu.async_copy(ztab_hbm.at[idx_v.at[j]], rows_v.at[j], sem)
                  for j in range(nch)]
        for cp in copies:
            cp.wait()
        pltpu.sync_copy(rows_v, out_hbm.at[wid])

    out = gk(ztab, idx3)
    return out.reshape(nidx, CHUNK_W)


def _final_body(cand_ref, cid_ref, idx_ref, thr_ref, scratch_ref, *, bm):
    ncand = K * CHUNK_W
    scratch_ref[...] = cand_ref[...]
    col = lax.broadcasted_iota(jnp.int32, (bm, ncand), 1)
    colk = lax.broadcasted_iota(jnp.int32, (bm, K), 1)
    ek = lax.broadcasted_iota(jnp.int32, (K, ncand), 0)
    ej = lax.broadcasted_iota(jnp.int32, (K, ncand), 1) // CHUNK_W
    onehot = jnp.where(ek == ej, 1.0, 0.0).astype(jnp.float32)
    cid_exp = jax.lax.dot_general(
        cid_ref[...].astype(jnp.float32), onehot,
        dimension_numbers=(((1,), (0,)), ((), ())),
        preferred_element_type=jnp.float32,
        precision=jax.lax.Precision.HIGHEST,
    )
    gidx = cid_exp.astype(jnp.int32) * CHUNK_W + (col % CHUNK_W)
    idxs = jnp.zeros((bm, K), jnp.int32)
    m = None
    for it in range(K):
        v = scratch_ref[...]
        m = jnp.max(v, axis=1, keepdims=True)
        eq = v == m
        sel = jnp.where(eq, gidx, 1 << 30)
        g = jnp.min(sel, axis=1, keepdims=True)
        idxs = jnp.where(colk == it, g, idxs)
        scratch_ref[...] = jnp.where(eq & (gidx == g), -1.0, v)
    idx_ref[...] = idxs
    thr_ref[...] = m


def _final_topk(cand, cid, bm):
    B = cand.shape[0]
    ncand = K * CHUNK_W
    grid = (B // bm,)
    return pl.pallas_call(
        functools.partial(_final_body, bm=bm),
        grid=grid,
        in_specs=[
            pl.BlockSpec((bm, ncand), lambda r: (r, 0)),
            pl.BlockSpec((bm, K), lambda r: (r, 0)),
        ],
        out_specs=[
            pl.BlockSpec((bm, K), lambda r: (r, 0)),
            pl.BlockSpec((bm, 1), lambda r: (r, 0)),
        ],
        out_shape=[
            jax.ShapeDtypeStruct((B, K), jnp.int32),
            jax.ShapeDtypeStruct((B, 1), jnp.float32),
        ],
        scratch_shapes=[pltpu.VMEM((bm, ncand), jnp.float32)],
    )(cand, cid)


def _decode_body(z_ref, w_ref, thr_ref, x_ref, bdec_ref,
                 ztopk_ref, xhat_ref, loss_ref, *, nsteps, scale):
    r = pl.program_id(0)
    n = pl.program_id(1)
    zt = z_ref[...]
    zt = jnp.where(zt >= thr_ref[...], zt, 0.0)
    ztopk_ref[...] = zt
    acc = jax.lax.dot_general(
        zt, w_ref[...],
        dimension_numbers=(((1,), (1,)), ((), ())),
        preferred_element_type=jnp.float32,
    )

    @pl.when(n == 0)
    def _():
        xhat_ref[...] = acc

    @pl.when(n != 0)
    def _():
        xhat_ref[...] += acc

    @pl.when(n == nsteps - 1)
    def _():
        xh = xhat_ref[...] + bdec_ref[...]
        xhat_ref[...] = xh
        diff = xh - x_ref[...]
        part = (jnp.sum(diff * diff) * scale).reshape(1, 1)

        @pl.when(r == 0)
        def _():
            loss_ref[...] = part

        @pl.when(r != 0)
        def _():
            loss_ref[...] += part


def _decode(z_pre, W_dec, thr, x, b_dec, bm, bn):
    B, D = x.shape
    N = z_pre.shape[1]
    nsteps = N // bn
    grid = (B // bm, nsteps)
    return pl.pallas_call(
        functools.partial(_decode_body, nsteps=nsteps, scale=1.0 / (B * D)),
        grid=grid,
        in_specs=[
            pl.BlockSpec((bm, bn), lambda r, n: (r, n)),
            pl.BlockSpec((D, bn), lambda r, n: (0, n)),
            pl.BlockSpec((bm, 1), lambda r, n: (r, 0)),
            pl.BlockSpec((bm, D), lambda r, n: (r, 0)),
            pl.BlockSpec((1, D), lambda r, n: (0, 0)),
        ],
        out_specs=[
            pl.BlockSpec((bm, bn), lambda r, n: (r, n)),
            pl.BlockSpec((bm, D), lambda r, n: (r, 0)),
            pl.BlockSpec((1, 1), lambda r, n: (0, 0)),
        ],
        out_shape=[
            jax.ShapeDtypeStruct((B, N), jnp.float32),
            jax.ShapeDtypeStruct((B, D), jnp.float32),
            jax.ShapeDtypeStruct((1, 1), jnp.float32),
        ],
    )(z_pre, W_dec, thr, x, b_dec.reshape(1, D))


def kernel(x, W_enc, b_enc, W_dec, b_dec):
    B, D = x.shape
    N = W_enc.shape[0]
    C = N // CHUNK_W

    z_pre, cm3 = _encode(x, W_enc, b_enc, b_dec, min(512, N))
    cm = jnp.transpose(cm3, (1, 0, 2)).reshape(B, C)
    cid, gat = _select(cm, min(1024, B))
    cand = _sc_gather(z_pre.reshape(B * C, CHUNK_W), gat.reshape(B * K))
    topk_idx, thr = _final_topk(cand.reshape(B, K * CHUNK_W), cid, min(512, B))
    z_topk, x_hat, loss = _decode(z_pre, W_dec, thr, x, b_dec,
                                  min(1024, B), min(512, N))
    return (x_hat, z_topk, z_pre, topk_idx, loss.reshape(()))
